# baseline (speedup 1.0000x reference)
import functools

import jax
import jax.numpy as jnp
from jax.experimental import pallas as pl
from jax.experimental.pallas import tpu as pltpu


def _fused_kernel(item_id_ref, uid_ref, x_ref, w_ih_ref, w_hh_ref,
                  b_ih_ref, b_hh_ref, item_ref, user_hbm_ref, w_out_ref,
                  b_out_ref, out_ref, ubuf_ref, sems, *, seq_len,
                  num_users):
    del item_id_ref
    n_sems = sems.shape[0]
    rows_per_sem = num_users // n_sems

    def issue(i, carry):
        pltpu.make_async_copy(user_hbm_ref.at[uid_ref[i]],
                              ubuf_ref.at[i],
                              sems.at[jax.lax.rem(i, n_sems)]).start()
        return carry

    jax.lax.fori_loop(0, num_users, issue, 0)

    x = x_ref[...].reshape(x_ref.shape[0], x_ref.shape[2])
    xb = x.astype(jnp.bfloat16)
    xr = (jnp.dot(xb, w_ih_ref[0].astype(jnp.bfloat16),
                  preferred_element_type=jnp.float32)
          + (b_ih_ref[0] + b_hh_ref[0]))
    xz = (jnp.dot(xb, w_ih_ref[1].astype(jnp.bfloat16),
                  preferred_element_type=jnp.float32)
          + (b_ih_ref[1] + b_hh_ref[1]))
    xn = (jnp.dot(xb, w_ih_ref[2].astype(jnp.bfloat16),
                  preferred_element_type=jnp.float32)
          + b_ih_ref[2])

    ur = w_hh_ref[0].astype(jnp.bfloat16)
    uz = w_hh_ref[1].astype(jnp.bfloat16)
    un = w_hh_ref[2].astype(jnp.bfloat16)
    b_hn = b_hh_ref[2]

    H = ur.shape[0]
    h = jnp.zeros((1, H), jnp.float32)
    h_sum = jnp.zeros((1, H), jnp.float32)

    for t in range(seq_len):
        hb = h.astype(jnp.bfloat16)
        hr = jnp.dot(hb, ur, preferred_element_type=jnp.float32)
        hz = jnp.dot(hb, uz, preferred_element_type=jnp.float32)
        hn = jnp.dot(hb, un, preferred_element_type=jnp.float32)
        r = jax.nn.sigmoid(xr[t:t + 1, :] + hr)
        z = jax.nn.sigmoid(xz[t:t + 1, :] + hz)
        n = jnp.tanh(xn[t:t + 1, :] + r * (hn + b_hn))
        h = n + z * (h - n)
        h_sum = h_sum + h

    mean_h = h_sum * (1.0 / float(seq_len))
    scale = item_ref[0] * mean_h

    for k in range(n_sems):
        pltpu.make_async_copy(ubuf_ref.at[pl.ds(0, rows_per_sem)],
                              ubuf_ref.at[pl.ds(0, rows_per_sem)],
                              sems.at[k]).wait()

    logits = jnp.broadcast_to(b_out_ref[...],
                              (num_users, b_out_ref.shape[-1]))
    for c in range(ubuf_ref.shape[1]):
        uc = ubuf_ref[:, c, :]
        sc = scale[:, 128 * c:128 * (c + 1)]
        logits = logits + jnp.dot(
            (uc * sc).astype(jnp.bfloat16),
            w_out_ref[c].astype(jnp.bfloat16),
            preferred_element_type=jnp.float32)
    m = jnp.max(logits, axis=-1, keepdims=True)
    e = jnp.exp(logits - m)
    out_ref[...] = e / jnp.sum(e, axis=-1, keepdims=True)


def kernel(item_table, user_table, w_ih, w_hh, b_ih, b_hh, w_out, b_out,
           item_id, user_ids, word_embeddings):
    seq_len, batch, emb_dim = word_embeddings.shape
    hidden = w_hh.shape[-1]
    rating_range = w_out.shape[-1]
    assert batch == 1 and hidden == emb_dim

    assert emb_dim % 128 == 0
    lane_chunks = emb_dim // 128
    num_users = user_ids.shape[0]
    item_idx = jnp.reshape(item_id, (1,))
    user3 = user_table.reshape(user_table.shape[0], lane_chunks, 128)
    w_out3 = w_out.reshape(lane_chunks, 128, rating_range)

    kern = functools.partial(_fused_kernel, seq_len=seq_len,
                             num_users=num_users)
    grid_spec = pltpu.PrefetchScalarGridSpec(
        num_scalar_prefetch=2,
        grid=(1,),
        in_specs=[
            pl.BlockSpec((seq_len, 1, emb_dim), lambda i, ii, uu: (0, 0, 0)),
            pl.BlockSpec((3, emb_dim, hidden), lambda i, ii, uu: (0, 0, 0)),
            pl.BlockSpec((3, hidden, hidden), lambda i, ii, uu: (0, 0, 0)),
            pl.BlockSpec((3, 1, hidden), lambda i, ii, uu: (0, 0, 0)),
            pl.BlockSpec((3, 1, hidden), lambda i, ii, uu: (0, 0, 0)),
            pl.BlockSpec((1, 1, emb_dim), lambda i, ii, uu: (ii[0], 0, 0)),
            pl.BlockSpec(memory_space=pl.ANY),
            pl.BlockSpec((lane_chunks, 128, rating_range),
                         lambda i, ii, uu: (0, 0, 0)),
            pl.BlockSpec((1, rating_range), lambda i, ii, uu: (0, 0)),
        ],
        out_specs=pl.BlockSpec((num_users, rating_range),
                               lambda i, ii, uu: (0, 0)),
        scratch_shapes=[
            pltpu.VMEM((num_users, lane_chunks, 128), jnp.float32),
            pltpu.SemaphoreType.DMA((8,)),
        ],
    )
    return pl.pallas_call(
        kern,
        out_shape=jax.ShapeDtypeStruct((num_users, rating_range),
                                       jnp.float32),
        grid_spec=grid_spec,
        compiler_params=pltpu.CompilerParams(
            dimension_semantics=("arbitrary",),
            disable_bounds_checks=True),
    )(item_idx, user_ids, word_embeddings, w_ih, w_hh, b_ih, b_hh,
      item_table.reshape(item_table.shape[0], 1, emb_dim),
      user3, w_out3, b_out)


# speedup vs baseline: 3.4388x; 3.4388x over previous
import functools

import jax
import jax.numpy as jnp
from jax.experimental import pallas as pl
from jax.experimental.pallas import tpu as pltpu


def _fused_kernel(item_id_ref, x_ref, w_ih_ref, w_hh_ref, b_ih_ref,
                  b_hh_ref, item_ref, user_ref, w_out_ref, b_out_ref,
                  out_ref, *, seq_len):
    del item_id_ref
    xb = x_ref[...]
    xr = (jnp.dot(xb, w_ih_ref[0], preferred_element_type=jnp.float32)
          + (b_ih_ref[0] + b_hh_ref[0]))
    xz = (jnp.dot(xb, w_ih_ref[1], preferred_element_type=jnp.float32)
          + (b_ih_ref[1] + b_hh_ref[1]))
    xn = (jnp.dot(xb, w_ih_ref[2], preferred_element_type=jnp.float32)
          + b_ih_ref[2])

    ur = w_hh_ref[0]
    uz = w_hh_ref[1]
    un = w_hh_ref[2]
    b_hn = b_hh_ref[2]

    H = ur.shape[0]
    h = jnp.zeros((1, H), jnp.float32)
    h_sum = jnp.zeros((1, H), jnp.float32)

    for t in range(seq_len):
        hb = h.astype(jnp.bfloat16)
        hr = jnp.dot(hb, ur, preferred_element_type=jnp.float32)
        hz = jnp.dot(hb, uz, preferred_element_type=jnp.float32)
        hn = jnp.dot(hb, un, preferred_element_type=jnp.float32)
        r = jax.nn.sigmoid(xr[t:t + 1, :] + hr)
        z = jax.nn.sigmoid(xz[t:t + 1, :] + hz)
        n = jnp.tanh(xn[t:t + 1, :] + r * (hn + b_hn))
        h = n + z * (h - n)
        h_sum = h_sum + h

    mean_h = h_sum * (1.0 / float(seq_len))

    scale = item_ref[0] * mean_h
    mul = (user_ref[...] * scale).astype(jnp.bfloat16)
    logits = (jnp.dot(mul, w_out_ref[...],
                      preferred_element_type=jnp.float32)
              + b_out_ref[...])
    m = jnp.max(logits, axis=-1, keepdims=True)
    e = jnp.exp(logits - m)
    out_ref[...] = e / jnp.sum(e, axis=-1, keepdims=True)


def kernel(item_table, user_table, w_ih, w_hh, b_ih, b_hh, w_out, b_out,
           item_id, user_ids, word_embeddings):
    seq_len, batch, emb_dim = word_embeddings.shape
    hidden = w_hh.shape[-1]
    rating_range = w_out.shape[-1]
    assert batch == 1 and hidden == emb_dim

    user_emb = user_table[jnp.asarray(user_ids)]
    num_users = user_emb.shape[0]
    item_idx = jnp.reshape(item_id, (1,))

    xb = word_embeddings.reshape(seq_len, emb_dim).astype(jnp.bfloat16)
    w_ih_b = w_ih.astype(jnp.bfloat16)
    w_hh_b = w_hh.astype(jnp.bfloat16)
    w_out_b = w_out.astype(jnp.bfloat16)

    kern = functools.partial(_fused_kernel, seq_len=seq_len)
    grid_spec = pltpu.PrefetchScalarGridSpec(
        num_scalar_prefetch=1,
        grid=(1,),
        in_specs=[
            pl.BlockSpec((seq_len, emb_dim), lambda i, ii: (0, 0)),
            pl.BlockSpec((3, emb_dim, hidden), lambda i, ii: (0, 0, 0)),
            pl.BlockSpec((3, hidden, hidden), lambda i, ii: (0, 0, 0)),
            pl.BlockSpec((3, 1, hidden), lambda i, ii: (0, 0, 0)),
            pl.BlockSpec((3, 1, hidden), lambda i, ii: (0, 0, 0)),
            pl.BlockSpec((1, 1, emb_dim), lambda i, ii: (ii[0], 0, 0)),
            pl.BlockSpec((num_users, emb_dim), lambda i, ii: (0, 0)),
            pl.BlockSpec((hidden, rating_range), lambda i, ii: (0, 0)),
            pl.BlockSpec((1, rating_range), lambda i, ii: (0, 0)),
        ],
        out_specs=pl.BlockSpec((num_users, rating_range),
                               lambda i, ii: (0, 0)),
    )
    return pl.pallas_call(
        kern,
        out_shape=jax.ShapeDtypeStruct((num_users, rating_range),
                                       jnp.float32),
        grid_spec=grid_spec,
        compiler_params=pltpu.CompilerParams(
            dimension_semantics=("arbitrary",)),
    )(item_idx, xb, w_ih_b, w_hh_b, b_ih, b_hh,
      item_table.reshape(item_table.shape[0], 1, emb_dim),
      user_emb, w_out_b, b_out)


# speedup vs baseline: 3.9427x; 1.1465x over previous
import functools

import jax
import jax.numpy as jnp
from jax.experimental import pallas as pl
from jax.experimental.pallas import tpu as pltpu


def _fused_kernel(x_ref, w_ih_ref, w_hh_ref, b_ih_ref,
                  b_hh_ref, item_ref, user_ref, w_out_ref, b_out_ref,
                  out_ref, *, seq_len):
    xb = x_ref[...]
    xr = (jnp.dot(xb, w_ih_ref[0], preferred_element_type=jnp.float32)
          + (b_ih_ref[0] + b_hh_ref[0]))
    xz = (jnp.dot(xb, w_ih_ref[1], preferred_element_type=jnp.float32)
          + (b_ih_ref[1] + b_hh_ref[1]))
    xn = (jnp.dot(xb, w_ih_ref[2], preferred_element_type=jnp.float32)
          + b_ih_ref[2])

    ur = w_hh_ref[0]
    uz = w_hh_ref[1]
    un = w_hh_ref[2]
    b_hn = b_hh_ref[2]

    H = ur.shape[0]
    h = jnp.zeros((1, H), jnp.float32)
    h_sum = jnp.zeros((1, H), jnp.float32)

    for t in range(seq_len):
        hb = h.astype(jnp.bfloat16)
        hr = jnp.dot(hb, ur, preferred_element_type=jnp.float32)
        hz = jnp.dot(hb, uz, preferred_element_type=jnp.float32)
        hn = jnp.dot(hb, un, preferred_element_type=jnp.float32)
        r = jax.nn.sigmoid(xr[t:t + 1, :] + hr)
        z = jax.nn.sigmoid(xz[t:t + 1, :] + hz)
        n = jnp.tanh(xn[t:t + 1, :] + r * (hn + b_hn))
        h = n + z * (h - n)
        h_sum = h_sum + h

    mean_h = h_sum * (1.0 / float(seq_len))

    scale = item_ref[...] * mean_h
    mul = (user_ref[...] * scale).astype(jnp.bfloat16)
    logits = (jnp.dot(mul, w_out_ref[...],
                      preferred_element_type=jnp.float32)
              + b_out_ref[...])
    m = jnp.max(logits, axis=-1, keepdims=True)
    e = jnp.exp(logits - m)
    out_ref[...] = e / jnp.sum(e, axis=-1, keepdims=True)


def kernel(item_table, user_table, w_ih, w_hh, b_ih, b_hh, w_out, b_out,
           item_id, user_ids, word_embeddings):
    seq_len, batch, emb_dim = word_embeddings.shape
    hidden = w_hh.shape[-1]
    rating_range = w_out.shape[-1]
    assert batch == 1 and hidden == emb_dim

    user_emb = user_table[jnp.asarray(user_ids)]
    num_users = user_emb.shape[0]

    xb = word_embeddings.reshape(seq_len, emb_dim).astype(jnp.bfloat16)
    w_ih_b = w_ih.astype(jnp.bfloat16)
    w_hh_b = w_hh.astype(jnp.bfloat16)
    w_out_b = w_out.astype(jnp.bfloat16)

    item_emb = item_table[item_id][None, :]
    kern = functools.partial(_fused_kernel, seq_len=seq_len)
    return pl.pallas_call(
        kern,
        out_shape=jax.ShapeDtypeStruct((num_users, rating_range),
                                       jnp.float32),
        compiler_params=pltpu.CompilerParams(
            dimension_semantics=()),
    )(xb, w_ih_b, w_hh_b, b_ih, b_hh, item_emb, user_emb, w_out_b, b_out)


# speedup vs baseline: 4.1038x; 1.0409x over previous
import functools

import jax
import jax.numpy as jnp
from jax.experimental import pallas as pl
from jax.experimental.pallas import tpu as pltpu


def _fused_kernel(x_ref, w_ih_ref, w_hh_ref, b_ih_ref,
                  b_hh_ref, item_ref, user_ref, w_out_ref, b_out_ref,
                  out_ref, *, seq_len):
    xb = x_ref[...]
    xr = (jnp.dot(xb, w_ih_ref[0], preferred_element_type=jnp.float32)
          + (b_ih_ref[0] + b_hh_ref[0]))
    xz = (jnp.dot(xb, w_ih_ref[1], preferred_element_type=jnp.float32)
          + (b_ih_ref[1] + b_hh_ref[1]))
    xn = (jnp.dot(xb, w_ih_ref[2], preferred_element_type=jnp.float32)
          + b_ih_ref[2])

    ur = w_hh_ref[0]
    uz = w_hh_ref[1]
    un = w_hh_ref[2]
    b_hn = b_hh_ref[2]

    H = ur.shape[0]
    h = jnp.zeros((1, H), jnp.float32)
    h_sum = jnp.zeros((1, H), jnp.float32)

    for t in range(seq_len):
        hb = h.astype(jnp.bfloat16)
        hr = jnp.dot(hb, ur, preferred_element_type=jnp.float32)
        hz = jnp.dot(hb, uz, preferred_element_type=jnp.float32)
        hn = jnp.dot(hb, un, preferred_element_type=jnp.float32)
        r = jax.nn.sigmoid(xr[t:t + 1, :] + hr)
        z = jax.nn.sigmoid(xz[t:t + 1, :] + hz)
        n = jnp.tanh(xn[t:t + 1, :] + r * (hn + b_hn))
        h = n + z * (h - n)
        h_sum = h_sum + h

    mean_h = h_sum * (1.0 / float(seq_len))

    scale = item_ref[...] * mean_h
    mul = user_ref[...] * scale
    logits = (jnp.dot(mul, w_out_ref[...],
                      preferred_element_type=jnp.float32)
              + b_out_ref[...])
    m = jnp.max(logits, axis=-1, keepdims=True)
    e = jnp.exp(logits - m)
    out_ref[...] = e / jnp.sum(e, axis=-1, keepdims=True)


def kernel(item_table, user_table, w_ih, w_hh, b_ih, b_hh, w_out, b_out,
           item_id, user_ids, word_embeddings):
    seq_len, batch, emb_dim = word_embeddings.shape
    hidden = w_hh.shape[-1]
    rating_range = w_out.shape[-1]
    assert batch == 1 and hidden == emb_dim

    user_emb = user_table[jnp.asarray(user_ids)]
    num_users = user_emb.shape[0]

    xb = word_embeddings.reshape(seq_len, emb_dim).astype(jnp.bfloat16)
    w_ih_b = w_ih.astype(jnp.bfloat16)
    w_hh_b = w_hh.astype(jnp.bfloat16)
    item_emb = item_table[item_id][None, :]
    kern = functools.partial(_fused_kernel, seq_len=seq_len)
    return pl.pallas_call(
        kern,
        out_shape=jax.ShapeDtypeStruct((num_users, rating_range),
                                       jnp.float32),
        compiler_params=pltpu.CompilerParams(
            dimension_semantics=()),
    )(xb, w_ih_b, w_hh_b, b_ih, b_hh, item_emb, user_emb, w_out, b_out)


# speedup vs baseline: 4.3737x; 1.0658x over previous
import functools

import jax
import jax.numpy as jnp
from jax.experimental import pallas as pl
from jax.experimental.pallas import tpu as pltpu


def _fused_kernel(x_ref, w_ref, b_ih_ref, b_hh_ref, item_ref,
                  user_hbm_ref, w_out_ref, b_out_ref, out_ref,
                  ubuf_ref, dma_sem, *, seq_len):
    cp = pltpu.make_async_copy(user_hbm_ref, ubuf_ref, dma_sem)
    cp.start()

    xb = x_ref[...]
    xr = (jnp.dot(xb, w_ref[0], preferred_element_type=jnp.float32)
          + (b_ih_ref[0] + b_hh_ref[0]))
    xz = (jnp.dot(xb, w_ref[1], preferred_element_type=jnp.float32)
          + (b_ih_ref[1] + b_hh_ref[1]))
    xn = (jnp.dot(xb, w_ref[2], preferred_element_type=jnp.float32)
          + b_ih_ref[2])

    ur = w_ref[3]
    uz = w_ref[4]
    un = w_ref[5]
    b_hn = b_hh_ref[2]

    H = ur.shape[0]
    h = jnp.zeros((1, H), jnp.float32)
    h_sum = jnp.zeros((1, H), jnp.float32)

    for t in range(seq_len):
        hb = h.astype(jnp.bfloat16)
        hr = jnp.dot(hb, ur, preferred_element_type=jnp.float32)
        hz = jnp.dot(hb, uz, preferred_element_type=jnp.float32)
        hn = jnp.dot(hb, un, preferred_element_type=jnp.float32)
        r = jax.nn.sigmoid(xr[t:t + 1, :] + hr)
        z = jax.nn.sigmoid(xz[t:t + 1, :] + hz)
        n = jnp.tanh(xn[t:t + 1, :] + r * (hn + b_hn))
        h = n + z * (h - n)
        h_sum = h_sum + h

    mean_h = h_sum * (1.0 / float(seq_len))
    scale = item_ref[...] * mean_h

    cp.wait()

    mul = ubuf_ref[...] * scale
    logits = (jnp.dot(mul, w_out_ref[...],
                      preferred_element_type=jnp.float32)
              + b_out_ref[...])
    m = jnp.max(logits, axis=-1, keepdims=True)
    e = jnp.exp(logits - m)
    out_ref[...] = e / jnp.sum(e, axis=-1, keepdims=True)


def kernel(item_table, user_table, w_ih, w_hh, b_ih, b_hh, w_out, b_out,
           item_id, user_ids, word_embeddings):
    seq_len, batch, emb_dim = word_embeddings.shape
    hidden = w_hh.shape[-1]
    rating_range = w_out.shape[-1]
    assert batch == 1 and hidden == emb_dim

    user_emb = user_table[jnp.asarray(user_ids)]
    num_users = user_emb.shape[0]

    xb = word_embeddings.reshape(seq_len, emb_dim).astype(jnp.bfloat16)
    w_all = jnp.concatenate([w_ih, w_hh], axis=0).astype(jnp.bfloat16)
    item_emb = item_table[item_id][None, :]

    kern = functools.partial(_fused_kernel, seq_len=seq_len)
    return pl.pallas_call(
        kern,
        out_shape=jax.ShapeDtypeStruct((num_users, rating_range),
                                       jnp.float32),
        in_specs=[
            pl.BlockSpec((seq_len, emb_dim), lambda: (0, 0)),
            pl.BlockSpec((6, emb_dim, hidden), lambda: (0, 0, 0)),
            pl.BlockSpec((3, 1, hidden), lambda: (0, 0, 0)),
            pl.BlockSpec((3, 1, hidden), lambda: (0, 0, 0)),
            pl.BlockSpec((1, emb_dim), lambda: (0, 0)),
            pl.BlockSpec(memory_space=pl.ANY),
            pl.BlockSpec((hidden, rating_range), lambda: (0, 0)),
            pl.BlockSpec((1, rating_range), lambda: (0, 0)),
        ],
        out_specs=pl.BlockSpec((num_users, rating_range), lambda: (0, 0)),
        scratch_shapes=[
            pltpu.VMEM((num_users, emb_dim), jnp.float32),
            pltpu.SemaphoreType.DMA,
        ],
        compiler_params=pltpu.CompilerParams(
            dimension_semantics=()),
    )(xb, w_all, b_ih, b_hh, item_emb, user_emb, w_out, b_out)


# speedup vs baseline: 4.4705x; 1.0221x over previous
import functools

import jax
import jax.numpy as jnp
from jax.experimental import pallas as pl
from jax.experimental.pallas import tpu as pltpu


def _fused_kernel(x_ref, w_ref, b_ih_ref, b_hh_ref, item_ref,
                  user_hbm_ref, w_out_ref, b_out_ref, out_ref,
                  ubuf_ref, dma_sem, *, seq_len):
    cp = pltpu.make_async_copy(user_hbm_ref, ubuf_ref, dma_sem)
    cp.start()

    wih_cat = jnp.concatenate([w_ref[0], w_ref[1], w_ref[2]], axis=1)
    whh_cat = jnp.concatenate([w_ref[3], w_ref[4], w_ref[5]], axis=1)
    b_cat = jnp.concatenate(
        [b_ih_ref[0] + b_hh_ref[0], b_ih_ref[1] + b_hh_ref[1],
         b_ih_ref[2]], axis=1)
    b_hn = b_hh_ref[2]
    H = w_ref.shape[2]

    xb = x_ref[...]
    xcat = (jnp.dot(xb, wih_cat, preferred_element_type=jnp.float32)
            + b_cat)

    h = jnp.zeros((1, H), jnp.float32)
    h_sum = jnp.zeros((1, H), jnp.float32)

    for t in range(seq_len):
        xt = xcat[t:t + 1, :]
        hh = jnp.dot(h.astype(jnp.bfloat16), whh_cat,
                     preferred_element_type=jnp.float32)
        rz = jax.nn.sigmoid(xt[:, :2 * H] + hh[:, :2 * H])
        r = rz[:, :H]
        z = rz[:, H:]
        n = jnp.tanh(xt[:, 2 * H:] + r * (hh[:, 2 * H:] + b_hn))
        h = n + z * (h - n)
        h_sum = h_sum + h

    mean_h = h_sum * (1.0 / float(seq_len))
    scale = item_ref[...] * mean_h

    cp.wait()

    mul = ubuf_ref[...] * scale
    logits = (jnp.dot(mul, w_out_ref[...],
                      preferred_element_type=jnp.float32)
              + b_out_ref[...])
    m = jnp.max(logits, axis=-1, keepdims=True)
    e = jnp.exp(logits - m)
    out_ref[...] = e / jnp.sum(e, axis=-1, keepdims=True)


def kernel(item_table, user_table, w_ih, w_hh, b_ih, b_hh, w_out, b_out,
           item_id, user_ids, word_embeddings):
    seq_len, batch, emb_dim = word_embeddings.shape
    hidden = w_hh.shape[-1]
    rating_range = w_out.shape[-1]
    assert batch == 1 and hidden == emb_dim

    user_emb = user_table[jnp.asarray(user_ids)]
    num_users = user_emb.shape[0]

    xb = word_embeddings.reshape(seq_len, emb_dim).astype(jnp.bfloat16)
    w_all = jnp.concatenate([w_ih, w_hh], axis=0).astype(jnp.bfloat16)
    item_emb = item_table[item_id][None, :]

    kern = functools.partial(_fused_kernel, seq_len=seq_len)
    return pl.pallas_call(
        kern,
        out_shape=jax.ShapeDtypeStruct((num_users, rating_range),
                                       jnp.float32),
        in_specs=[
            pl.BlockSpec((seq_len, emb_dim), lambda: (0, 0)),
            pl.BlockSpec((6, emb_dim, hidden), lambda: (0, 0, 0)),
            pl.BlockSpec((3, 1, hidden), lambda: (0, 0, 0)),
            pl.BlockSpec((3, 1, hidden), lambda: (0, 0, 0)),
            pl.BlockSpec((1, emb_dim), lambda: (0, 0)),
            pl.BlockSpec(memory_space=pl.ANY),
            pl.BlockSpec((hidden, rating_range), lambda: (0, 0)),
            pl.BlockSpec((1, rating_range), lambda: (0, 0)),
        ],
        out_specs=pl.BlockSpec((num_users, rating_range), lambda: (0, 0)),
        scratch_shapes=[
            pltpu.VMEM((num_users, emb_dim), jnp.float32),
            pltpu.SemaphoreType.DMA,
        ],
        compiler_params=pltpu.CompilerParams(
            dimension_semantics=()),
    )(xb, w_all, b_ih, b_hh, item_emb, user_emb, w_out, b_out)
